# baseline (speedup 1.0000x reference)
import functools

import jax
import jax.numpy as jnp
from jax import lax
from jax.experimental import pallas as pl
from jax.experimental.pallas import tpu as pltpu
from jax.experimental.pallas import tpu_sc as plsc

NC = 2
NS = 16
NW = NC * NS


def _sc_mesh():
    return plsc.VectorSubcoreMesh(
        core_axis_name="c", subcore_axis_name="s",
        num_cores=NC, num_subcores=NS)


def _make_degree(NPD, C, K):
    ept = NPD // NS
    W = 8

    @functools.partial(
        pl.kernel,
        out_type=jax.ShapeDtypeStruct((NC, 2, NPD), jnp.float32),
        mesh=_sc_mesh(),
        scratch_types=[
            pltpu.VMEM((C, K), jnp.int32),
            pltpu.VMEM((C, K), jnp.int32),
            pltpu.VMEM((K,), jnp.float32),
            pltpu.VMEM((128,), jnp.float32),
            pltpu.VMEM_SHARED((NPD,), jnp.float32),
            pltpu.VMEM_SHARED((NPD,), jnp.float32),
            pltpu.SemaphoreType.DMA,
        ],
        compiler_params=pltpu.CompilerParams(use_tc_tiling_on_sc=False),
    )
    def kern(graph_r, out, src_v, dst_v, ones, zb, acc_s, acc_d, sem):
        c = lax.axis_index("c")
        s = lax.axis_index("s")
        wid = s * NC + c
        for i in range(8):
            zb[pl.ds(i * 16, 16)] = jnp.zeros((16,), jnp.float32)
        for i in range(K // 16):
            ones[pl.ds(i * 16, 16)] = jnp.ones((16,), jnp.float32)
        for k in range(ept // 128):
            pltpu.sync_copy(zb, acc_s.at[pl.ds(s * ept + k * 128, 128)])
            pltpu.sync_copy(zb, acc_d.at[pl.ds(s * ept + k * 128, 128)])
        pltpu.sync_copy(graph_r.at[0].at[wid], src_v)
        pltpu.sync_copy(graph_r.at[1].at[wid], dst_v)
        plsc.subcore_barrier()

        def step_s(j, carry):
            pltpu.async_copy(ones, acc_s.at[src_v.at[j]], sem, add=True)

            @pl.when(j >= W)
            def _():
                pltpu.make_async_copy(ones, acc_s.at[src_v.at[j]], sem).wait()

            return carry

        def step_d(j, carry):
            pltpu.async_copy(ones, acc_d.at[dst_v.at[j]], sem, add=True)
            pltpu.make_async_copy(ones, acc_d.at[dst_v.at[j]], sem).wait()
            return carry

        lax.fori_loop(0, C, step_s, 0)
        lax.fori_loop(0, C, step_d, 0)
        for u in range(W):
            pltpu.make_async_copy(ones, acc_d.at[dst_v.at[u]], sem).wait()
        plsc.subcore_barrier()
        pltpu.sync_copy(acc_s.at[pl.ds(s * ept, ept)],
                        out.at[c].at[0].at[pl.ds(s * ept, ept)])
        pltpu.sync_copy(acc_d.at[pl.ds(s * ept, ept)],
                        out.at[c].at[1].at[pl.ds(s * ept, ept)])

    return kern


def _make_edge_pass(NB, D, C, K):
    rpt = NB // NS
    R = 8
    P = 4
    NR = C // R
    TAIL = C - NR * R
    assert 0 < TAIL < R or TAIL == 0

    @functools.partial(
        pl.kernel,
        out_type=jax.ShapeDtypeStruct((NC, NB, D), jnp.bfloat16),
        mesh=_sc_mesh(),
        scratch_types=(
            [pltpu.VMEM((C, K), jnp.int32),
             pltpu.VMEM((C, K), jnp.int32)]
            + [pltpu.VMEM((K, D), jnp.bfloat16)] * R
            + [pltpu.SemaphoreType.DMA] * R
            + [pltpu.VMEM_SHARED((NB, D), jnp.bfloat16)]
        ),
        compiler_params=pltpu.CompilerParams(use_tc_tiling_on_sc=False),
    )
    def kern(table, graph_r, out, src_v, dst_v, *rest):
        bufs = rest[:R]
        sems = rest[R:2 * R]
        acc = rest[2 * R]
        c = lax.axis_index("c")
        s = lax.axis_index("s")
        wid = s * NC + c
        zb = bufs[0]

        def zrow(r, carry):
            for i in range(D // 32):
                zb[r, pl.ds(i * 32, 32)] = jnp.zeros((32,), jnp.bfloat16)
            return carry

        lax.fori_loop(0, K, zrow, 0)
        off = 0
        while off < rpt:
            n = min(K, rpt - off)
            pltpu.sync_copy(zb.at[pl.ds(0, n)],
                            acc.at[pl.ds(s * rpt + off, n)])
            off += n
        pltpu.sync_copy(graph_r.at[0].at[wid], src_v)
        pltpu.sync_copy(graph_r.at[1].at[wid], dst_v)
        plsc.subcore_barrier()

        for r in range(P):
            pltpu.async_copy(table.at[src_v.at[r]], bufs[r], sems[r])

        def slot(j, u, static):
            q = (u + P) % R
            pltpu.make_async_copy(
                table.at[src_v.at[j]], bufs[u], sems[u]).wait()
            pltpu.async_copy(
                bufs[u], acc.at[dst_v.at[j]], sems[u], add=True)

            def retire():
                pltpu.make_async_copy(
                    bufs[q], acc.at[dst_v.at[j]], sems[q]).wait()

            def prefetch():
                pltpu.async_copy(
                    table.at[src_v.at[j + P]], bufs[q], sems[q])

            if static:
                if j + P >= R:
                    retire()
                if j + P < C:
                    prefetch()
            else:
                pl.when(j + P >= R)(retire)
                pl.when(j + P < C)(prefetch)

        def round_(i, carry):
            for u in range(R):
                slot(i * R + u, u, False)
            return carry

        lax.fori_loop(0, NR, round_, 0)
        for u in range(TAIL):
            slot(NR * R + u, u, True)
        for u in range(P):
            q = (C - P + u) % R
            pltpu.make_async_copy(
                bufs[q], acc.at[dst_v.at[C - P + u]], sems[q]).wait()
        plsc.subcore_barrier()
        off = 0
        while off < rpt:
            n = min(K, rpt - off)
            pltpu.sync_copy(acc.at[pl.ds(s * rpt + off, n)],
                            out.at[c].at[pl.ds(s * rpt + off, n)])
            off += n

    return kern


def _tc_matmul(x, W1):
    DH = W1.shape[1]
    N = x.shape[0]

    def body(x_ref, w_ref, o_ref):
        o_ref[...] = jnp.dot(x_ref[...], w_ref[...],
                             preferred_element_type=jnp.float32)

    return pl.pallas_call(
        body, out_shape=jax.ShapeDtypeStruct((N, DH), jnp.float32),
    )(x, W1)


def _tc_scale(y, deg_t, N, NB, NPD):
    DH = y.shape[1]

    def body(y_ref, d_ref, o_ref):
        deg = jnp.sum(d_ref[...], axis=1, keepdims=True)
        nsrc = lax.rsqrt(jnp.maximum(deg[:N], 1.0))
        t = (y_ref[...] * nsrc).astype(jnp.bfloat16)
        o_ref[...] = jnp.concatenate(
            [t, jnp.zeros((NB - N, DH), jnp.bfloat16)], axis=0)

    return pl.pallas_call(
        body, out_shape=jax.ShapeDtypeStruct((NB, DH), jnp.bfloat16),
    )(y, deg_t)


def _tc_norms(deg_t, N, NPD, DH):
    def body(d_ref, o_ref):
        deg = jnp.sum(d_ref[...], axis=1, keepdims=True)
        nsrc = lax.rsqrt(jnp.maximum(deg[:N], 1.0))
        ndst = lax.rsqrt(jnp.maximum(deg[NPD:NPD + N], 1.0))
        o_ref[...] = jnp.concatenate(
            [jnp.broadcast_to(nsrc, (N, DH)),
             jnp.broadcast_to(ndst, (N, DH))], axis=1)

    return pl.pallas_call(
        body, out_shape=jax.ShapeDtypeStruct((N, 2 * DH), jnp.float32),
    )(deg_t)


def _tc_mid(aggp, normb, W2p, b1r, N, NB, NPD):
    D2 = W2p.shape[1]

    def body(a_ref, d_ref, w_ref, b_ref, o_ref):
        nsrc = d_ref[:, 0:1]
        ndst = d_ref[:, D2:D2 + 1]
        a = (a_ref[0].astype(jnp.float32)
             + a_ref[1].astype(jnp.float32))[:N]
        h = jnp.maximum(a * ndst + b_ref[...], 0.0)
        t = (jnp.dot(h, w_ref[...], preferred_element_type=jnp.float32)
             * nsrc).astype(jnp.bfloat16)
        o_ref[...] = jnp.concatenate(
            [t, jnp.zeros((NB - N, D2), jnp.bfloat16)], axis=0)

    return pl.pallas_call(
        body, out_shape=jax.ShapeDtypeStruct((NB, D2), jnp.bfloat16),
    )(aggp, normb, W2p, b1r)


def _tc_post(aggp, normb, b2r, N, NB, NPD, NCLS, DH):
    D2 = b2r.shape[1]

    def body(a_ref, d_ref, b_ref, o_ref):
        ndst = d_ref[:, DH:DH + 1]
        a = (a_ref[0].astype(jnp.float32)
             + a_ref[1].astype(jnp.float32))[:N]
        o_ref[...] = jnp.maximum(a * ndst + b_ref[...], 0.0)[:, :NCLS]

    return pl.pallas_call(
        body, out_shape=jax.ShapeDtypeStruct((N, NCLS), jnp.float32),
    )(aggp, normb, b2r)


def kernel(graph, x, W1, b1, W2, b2):
    N, D_IN = x.shape
    E = graph.shape[1]
    DH = W1.shape[1]
    NCLS = W2.shape[1]
    D2 = DH
    NB = N + 16
    NPD = ((N + 16 * NS - 1) // (16 * NS)) * (16 * NS)

    K = 256
    C = -(-E // (NW * K))
    EP = NW * C * K

    g = graph.astype(jnp.int32)
    pad_idx = (N + (jnp.arange(EP - E, dtype=jnp.int32) % 16))[None, :]
    graph_r = jnp.concatenate(
        [g, jnp.broadcast_to(pad_idx, (2, EP - E))], axis=1,
    ).reshape(2, NW, C, K)

    W2p = jnp.pad(W2, ((0, 0), (0, D2 - NCLS)))
    b1r = b1.reshape(1, DH)
    b2r = jnp.pad(b2, (0, D2 - NCLS)).reshape(1, D2)

    degp = _make_degree(NPD, C, K)(graph_r)
    deg_t = degp.reshape(2, 2 * NPD).T
    y1 = _tc_matmul(x, W1)
    t1 = _tc_scale(y1, deg_t, N, NB, NPD)
    normb = _tc_norms(deg_t, N, NPD, DH)
    agg1p = _make_edge_pass(NB, DH, C, K)(t1, graph_r)
    t2 = _tc_mid(agg1p, normb, W2p, b1r, N, NB, NPD)
    agg2p = _make_edge_pass(NB, D2, C, K)(t2, graph_r)
    return _tc_post(agg2p, normb, b2r, N, NB, NPD, NCLS, DH)


# speedup vs baseline: 1.1405x; 1.1405x over previous
import functools

import jax
import jax.numpy as jnp
from jax import lax
from jax.experimental import pallas as pl
from jax.experimental.pallas import tpu as pltpu
from jax.experimental.pallas import tpu_sc as plsc

NC = 2
NS = 16
NW = NC * NS


def _sc_mesh():
    return plsc.VectorSubcoreMesh(
        core_axis_name="c", subcore_axis_name="s",
        num_cores=NC, num_subcores=NS)


def _make_degree(NPD, C, K):
    ept = NPD // NS
    W = 8

    @functools.partial(
        pl.kernel,
        out_type=jax.ShapeDtypeStruct((NC, 2, NPD), jnp.float32),
        mesh=_sc_mesh(),
        scratch_types=[
            pltpu.VMEM((C, K), jnp.int32),
            pltpu.VMEM((C, K), jnp.int32),
            pltpu.VMEM((K,), jnp.float32),
            pltpu.VMEM((128,), jnp.float32),
            pltpu.VMEM_SHARED((NPD,), jnp.float32),
            pltpu.VMEM_SHARED((NPD,), jnp.float32),
            pltpu.SemaphoreType.DMA,
        ],
        compiler_params=pltpu.CompilerParams(use_tc_tiling_on_sc=False),
    )
    def kern(graph_r, out, src_v, dst_v, ones, zb, acc_s, acc_d, sem):
        c = lax.axis_index("c")
        s = lax.axis_index("s")
        wid = s * NC + c
        for i in range(8):
            zb[pl.ds(i * 16, 16)] = jnp.zeros((16,), jnp.float32)
        for i in range(K // 16):
            ones[pl.ds(i * 16, 16)] = jnp.ones((16,), jnp.float32)
        for k in range(ept // 128):
            pltpu.sync_copy(zb, acc_s.at[pl.ds(s * ept + k * 128, 128)])
            pltpu.sync_copy(zb, acc_d.at[pl.ds(s * ept + k * 128, 128)])
        pltpu.sync_copy(graph_r.at[0].at[wid], src_v)
        pltpu.sync_copy(graph_r.at[1].at[wid], dst_v)
        plsc.subcore_barrier()

        def step_s(j, carry):
            pltpu.async_copy(ones, acc_s.at[src_v.at[j]], sem, add=True)

            @pl.when(j >= W)
            def _():
                pltpu.make_async_copy(ones, acc_s.at[src_v.at[j]], sem).wait()

            return carry

        def step_d(j, carry):
            pltpu.async_copy(ones, acc_d.at[dst_v.at[j]], sem, add=True)
            pltpu.make_async_copy(ones, acc_d.at[dst_v.at[j]], sem).wait()
            return carry

        lax.fori_loop(0, C, step_s, 0)
        lax.fori_loop(0, C, step_d, 0)
        for u in range(W):
            pltpu.make_async_copy(ones, acc_d.at[dst_v.at[u]], sem).wait()
        plsc.subcore_barrier()
        pltpu.sync_copy(acc_s.at[pl.ds(s * ept, ept)],
                        out.at[c].at[0].at[pl.ds(s * ept, ept)])
        pltpu.sync_copy(acc_d.at[pl.ds(s * ept, ept)],
                        out.at[c].at[1].at[pl.ds(s * ept, ept)])

    return kern


def _make_edge_pass(NB, D, C, K):
    rpt = NB // NS
    R = 12
    P = 6
    NR = C // R
    TAIL = C - NR * R
    assert 0 < TAIL < R or TAIL == 0

    @functools.partial(
        pl.kernel,
        out_type=jax.ShapeDtypeStruct((NC, NB, D), jnp.bfloat16),
        mesh=_sc_mesh(),
        scratch_types=(
            [pltpu.VMEM((C, K), jnp.int32),
             pltpu.VMEM((C, K), jnp.int32)]
            + [pltpu.VMEM((K, D), jnp.bfloat16)] * R
            + [pltpu.SemaphoreType.DMA] * R
            + [pltpu.VMEM_SHARED((NB, D), jnp.bfloat16)]
        ),
        compiler_params=pltpu.CompilerParams(use_tc_tiling_on_sc=False),
    )
    def kern(table, graph_r, out, src_v, dst_v, *rest):
        bufs = rest[:R]
        sems = rest[R:2 * R]
        acc = rest[2 * R]
        c = lax.axis_index("c")
        s = lax.axis_index("s")
        wid = s * NC + c
        zb = bufs[0]

        def zrow(r, carry):
            for i in range(D // 32):
                zb[r, pl.ds(i * 32, 32)] = jnp.zeros((32,), jnp.bfloat16)
            return carry

        lax.fori_loop(0, K, zrow, 0)
        off = 0
        while off < rpt:
            n = min(K, rpt - off)
            pltpu.sync_copy(zb.at[pl.ds(0, n)],
                            acc.at[pl.ds(s * rpt + off, n)])
            off += n
        pltpu.sync_copy(graph_r.at[0].at[wid], src_v)
        pltpu.sync_copy(graph_r.at[1].at[wid], dst_v)
        plsc.subcore_barrier()

        for r in range(P):
            pltpu.async_copy(table.at[src_v.at[r]], bufs[r], sems[r])

        def slot(j, u, static):
            q = (u + P) % R
            pltpu.make_async_copy(
                table.at[src_v.at[j]], bufs[u], sems[u]).wait()
            pltpu.async_copy(
                bufs[u], acc.at[dst_v.at[j]], sems[u], add=True)

            def retire():
                pltpu.make_async_copy(
                    bufs[q], acc.at[dst_v.at[j]], sems[q]).wait()

            def prefetch():
                pltpu.async_copy(
                    table.at[src_v.at[j + P]], bufs[q], sems[q])

            if static:
                if j + P >= R:
                    retire()
                if j + P < C:
                    prefetch()
            else:
                pl.when(j + P >= R)(retire)
                pl.when(j + P < C)(prefetch)

        def round_(i, carry):
            for u in range(R):
                slot(i * R + u, u, False)
            return carry

        lax.fori_loop(0, NR, round_, 0)
        for u in range(TAIL):
            slot(NR * R + u, u, True)
        for u in range(P):
            q = (C - P + u) % R
            pltpu.make_async_copy(
                bufs[q], acc.at[dst_v.at[C - P + u]], sems[q]).wait()
        plsc.subcore_barrier()
        off = 0
        while off < rpt:
            n = min(K, rpt - off)
            pltpu.sync_copy(acc.at[pl.ds(s * rpt + off, n)],
                            out.at[c].at[pl.ds(s * rpt + off, n)])
            off += n

    return kern


def _tc_matmul(x, W1):
    DH = W1.shape[1]
    N = x.shape[0]

    def body(x_ref, w_ref, o_ref):
        o_ref[...] = jnp.dot(x_ref[...], w_ref[...],
                             preferred_element_type=jnp.float32)

    return pl.pallas_call(
        body, out_shape=jax.ShapeDtypeStruct((N, DH), jnp.float32),
    )(x, W1)


def _tc_scale(y, deg_t, N, NB, NPD):
    DH = y.shape[1]

    def body(y_ref, d_ref, o_ref):
        deg = jnp.sum(d_ref[...], axis=1, keepdims=True)
        nsrc = lax.rsqrt(jnp.maximum(deg[:N], 1.0))
        t = (y_ref[...] * nsrc).astype(jnp.bfloat16)
        o_ref[...] = jnp.concatenate(
            [t, jnp.zeros((NB - N, DH), jnp.bfloat16)], axis=0)

    return pl.pallas_call(
        body, out_shape=jax.ShapeDtypeStruct((NB, DH), jnp.bfloat16),
    )(y, deg_t)


def _tc_norms(deg_t, N, NPD, DH):
    def body(d_ref, o_ref):
        deg = jnp.sum(d_ref[...], axis=1, keepdims=True)
        nsrc = lax.rsqrt(jnp.maximum(deg[:N], 1.0))
        ndst = lax.rsqrt(jnp.maximum(deg[NPD:NPD + N], 1.0))
        o_ref[...] = jnp.concatenate(
            [jnp.broadcast_to(nsrc, (N, DH)),
             jnp.broadcast_to(ndst, (N, DH))], axis=1)

    return pl.pallas_call(
        body, out_shape=jax.ShapeDtypeStruct((N, 2 * DH), jnp.float32),
    )(deg_t)


def _tc_mid(aggp, normb, W2p, b1r, N, NB, NPD):
    D2 = W2p.shape[1]

    def body(a_ref, d_ref, w_ref, b_ref, o_ref):
        nsrc = d_ref[:, 0:1]
        ndst = d_ref[:, D2:D2 + 1]
        a = (a_ref[0].astype(jnp.float32)
             + a_ref[1].astype(jnp.float32))[:N]
        h = jnp.maximum(a * ndst + b_ref[...], 0.0)
        t = (jnp.dot(h, w_ref[...], preferred_element_type=jnp.float32)
             * nsrc).astype(jnp.bfloat16)
        o_ref[...] = jnp.concatenate(
            [t, jnp.zeros((NB - N, D2), jnp.bfloat16)], axis=0)

    return pl.pallas_call(
        body, out_shape=jax.ShapeDtypeStruct((NB, D2), jnp.bfloat16),
    )(aggp, normb, W2p, b1r)


def _tc_post(aggp, normb, b2r, N, NB, NPD, NCLS, DH):
    D2 = b2r.shape[1]

    def body(a_ref, d_ref, b_ref, o_ref):
        ndst = d_ref[:, DH:DH + 1]
        a = (a_ref[0].astype(jnp.float32)
             + a_ref[1].astype(jnp.float32))[:N]
        o_ref[...] = jnp.maximum(a * ndst + b_ref[...], 0.0)[:, :NCLS]

    return pl.pallas_call(
        body, out_shape=jax.ShapeDtypeStruct((N, NCLS), jnp.float32),
    )(aggp, normb, b2r)


def kernel(graph, x, W1, b1, W2, b2):
    N, D_IN = x.shape
    E = graph.shape[1]
    DH = W1.shape[1]
    NCLS = W2.shape[1]
    D2 = DH
    NB = N + 16
    NPD = ((N + 16 * NS - 1) // (16 * NS)) * (16 * NS)

    K = 128
    C = -(-E // (NW * K))
    EP = NW * C * K

    g = graph.astype(jnp.int32)
    pad_idx = (N + (jnp.arange(EP - E, dtype=jnp.int32) % 16))[None, :]
    graph_r = jnp.concatenate(
        [g, jnp.broadcast_to(pad_idx, (2, EP - E))], axis=1,
    ).reshape(2, NW, C, K)

    W2p = jnp.pad(W2, ((0, 0), (0, D2 - NCLS)))
    b1r = b1.reshape(1, DH)
    b2r = jnp.pad(b2, (0, D2 - NCLS)).reshape(1, D2)

    degp = _make_degree(NPD, C, K)(graph_r)
    deg_t = degp.reshape(2, 2 * NPD).T
    y1 = _tc_matmul(x, W1)
    t1 = _tc_scale(y1, deg_t, N, NB, NPD)
    normb = _tc_norms(deg_t, N, NPD, DH)
    agg1p = _make_edge_pass(NB, DH, C, K)(t1, graph_r)
    t2 = _tc_mid(agg1p, normb, W2p, b1r, N, NB, NPD)
    agg2p = _make_edge_pass(NB, D2, C, K)(t2, graph_r)
    return _tc_post(agg2p, normb, b2r, N, NB, NPD, NCLS, DH)
